# Initial kernel scaffold; baseline (speedup 1.0000x reference)
import jax
import jax.numpy as jnp
from jax.experimental import pallas as pl


def kernel(user_emb, item_emb, edge_index, edge_weight, users):
    raise NotImplementedError("write your pallas kernel here")


# speedup vs baseline: 1.7741x; 1.7741x over previous
import functools

import jax
import jax.numpy as jnp
from jax import lax
from jax.experimental import pallas as pl
from jax.experimental.pallas import tpu as pltpu
from jax.experimental.pallas import tpu_sc as plsc

NUM_USERS = 25000
NUM_ITEMS = 25000
N = NUM_USERS + NUM_ITEMS
NP = 50176
HALF = NP // 2
ACC_ROWS = 25600
ZCHUNK = 80
WCHUNK = 112
E = 800000
D = 64
N_LAYERS = 3
B = 1024
ECHUNK = 80
EPT = E // 16
NCHUNK = EPT // ECHUNK

_mesh = plsc.VectorSubcoreMesh(core_axis_name="c", subcore_axis_name="s")
_sc_params = pltpu.CompilerParams(
    needs_layout_passes=False, use_tc_tiling_on_sc=False)


@functools.partial(
    pl.kernel,
    out_type=(
        jax.ShapeDtypeStruct((NP, D), jnp.float32),
        jax.ShapeDtypeStruct((NP, D), jnp.float32),
    ),
    mesh=_mesh,
    scratch_types=[
        pltpu.VMEM((ECHUNK,), jnp.int32),
        pltpu.VMEM((ECHUNK,), jnp.int32),
        pltpu.VMEM((ECHUNK,), jnp.int32),
        pltpu.VMEM((ECHUNK,), jnp.float32),
        pltpu.VMEM((ECHUNK, D), jnp.float32),
        pltpu.VMEM((WCHUNK, D), jnp.float32),
        pltpu.VMEM((WCHUNK, D), jnp.float32),
        pltpu.VMEM_SHARED((ACC_ROWS, D), jnp.float32),
        pltpu.SemaphoreType.DMA,
    ],
    compiler_params=_sc_params,
)
def _layer(x_hbm, src_hbm, dst_hbm, w_hbm, s_hbm, y_hbm, sout_hbm,
           sidx, didx, dloc, wbuf, rows, stga, stgb, acc, sem):
    sc = lax.axis_index("c")
    sub = lax.axis_index("s")
    zero16 = jnp.zeros((16,), jnp.float32)

    def zrow(i, _):
        for q in range(4):
            stga[i, pl.ds(q * 16, 16)] = zero16
        return 0
    lax.fori_loop(0, ZCHUNK, zrow, 0)
    def zcopy(c, _):
        pltpu.sync_copy(stga.at[pl.ds(0, ZCHUNK)],
                        acc.at[pl.ds(sub * 1600 + c * ZCHUNK, ZCHUNK)])
        return 0
    lax.fori_loop(0, 20, zcopy, 0)
    plsc.subcore_barrier()

    half_base = sc * HALF

    def chunk(c, _):
        base = sub * EPT + c * ECHUNK
        pltpu.sync_copy(src_hbm.at[pl.ds(base, ECHUNK)], sidx)
        pltpu.sync_copy(dst_hbm.at[pl.ds(base, ECHUNK)], didx)
        pltpu.sync_copy(w_hbm.at[pl.ds(base, ECHUNK)], wbuf)
        pltpu.async_copy(x_hbm.at[sidx], rows, sem).wait()
        for i in range(ECHUNK // 16):
            sl = pl.ds(i * 16, 16)
            t = didx[sl] - half_base
            ok = (t >= 0) & (t < HALF)
            dloc[sl] = jnp.where(ok, t, HALF)
        def edge(e, _):
            w16 = plsc.load_gather(wbuf, [jnp.full((16,), e, jnp.int32)])
            for q in range(4):
                sl = pl.ds(q * 16, 16)
                rows[e, sl] = rows[e, sl] * w16
            return 0
        lax.fori_loop(0, ECHUNK, edge, 0)
        pltpu.sync_copy(rows, acc.at[dloc], add=True)
        return 0
    lax.fori_loop(0, NCHUNK, chunk, 0)
    plsc.subcore_barrier()

    for c in range(14):
        lb = sub * 1568 + c * WCHUNK
        gb = half_base + lb
        pltpu.sync_copy(acc.at[pl.ds(lb, WCHUNK)], stga.at[pl.ds(0, WCHUNK)])
        pltpu.sync_copy(s_hbm.at[pl.ds(gb, WCHUNK)], stgb.at[pl.ds(0, WCHUNK)])
        def srow(i, _):
            for q in range(4):
                sl = pl.ds(q * 16, 16)
                stgb[i, sl] = stgb[i, sl] + stga[i, sl]
            return 0
        lax.fori_loop(0, WCHUNK, srow, 0)
        pltpu.sync_copy(stga.at[pl.ds(0, WCHUNK)], y_hbm.at[pl.ds(gb, WCHUNK)])
        pltpu.sync_copy(stgb.at[pl.ds(0, WCHUNK)], sout_hbm.at[pl.ds(gb, WCHUNK)])


@functools.partial(
    pl.kernel,
    out_type=jax.ShapeDtypeStruct((B, D), jnp.float32),
    mesh=_mesh,
    scratch_types=[
        pltpu.VMEM((B // 32,), jnp.int32),
        pltpu.VMEM((B // 32, D), jnp.float32),
        pltpu.SemaphoreType.DMA,
    ],
    compiler_params=_sc_params,
)
def _gather_users(s_hbm, users_hbm, out_hbm, idxv, rowsv, sem):
    wid = lax.axis_index("s") * 2 + lax.axis_index("c")
    base = wid * (B // 32)
    pltpu.sync_copy(users_hbm.at[pl.ds(base, B // 32)], idxv)
    pltpu.async_copy(s_hbm.at[idxv], rowsv, sem).wait()
    pltpu.sync_copy(rowsv, out_hbm.at[pl.ds(base, B // 32)])


ITEM_BLK = 512
ITEMS_PAD = 25088


def _rating_body(u_ref, i_ref, o_ref):
    acc = lax.dot_general(u_ref[...], i_ref[...],
                          (((1,), (1,)), ((), ())),
                          preferred_element_type=jnp.float32)
    o_ref[...] = jax.nn.sigmoid(acc * (1.0 / (N_LAYERS + 1) ** 2))


_rating = pl.pallas_call(
    _rating_body,
    out_shape=jax.ShapeDtypeStruct((B, ITEMS_PAD), jnp.float32),
    grid=(ITEMS_PAD // ITEM_BLK,),
    in_specs=[
        pl.BlockSpec((B, D), lambda j: (0, 0)),
        pl.BlockSpec((ITEM_BLK, D), lambda j: (j, 0)),
    ],
    out_specs=pl.BlockSpec((B, ITEM_BLK), lambda j: (0, j)),
)


def kernel(user_emb, item_emb, edge_index, edge_weight, users):
    x0 = jnp.concatenate(
        [user_emb, item_emb,
         jnp.zeros((NP - N, D), jnp.float32)], axis=0)
    src = edge_index[0]
    dst = edge_index[1]
    x, s = x0, x0
    for _ in range(N_LAYERS):
        x, s = _layer(x, src, dst, edge_weight, s)
    u_sel = _gather_users(s, users)
    items = jnp.concatenate(
        [s[NUM_USERS:NUM_USERS + NUM_ITEMS],
         jnp.zeros((ITEMS_PAD - NUM_ITEMS, D), jnp.float32)], axis=0)
    rating = _rating(u_sel, items)
    return rating[:, :NUM_ITEMS]


# speedup vs baseline: 3.3823x; 1.9065x over previous
import functools

import jax
import jax.numpy as jnp
from jax import lax
from jax.experimental import pallas as pl
from jax.experimental.pallas import tpu as pltpu
from jax.experimental.pallas import tpu_sc as plsc

NUM_USERS = 25000
NUM_ITEMS = 25000
N = NUM_USERS + NUM_ITEMS
NP = 50176
HALF = NP // 2
ACC_ROWS = 25216
D = 64
E = 800000
N_LAYERS = 3
B = 1024

EPT = E // 16
SUP = 2000
NSUP = EPT // SUP
NGRP = SUP // 16
CAP = SUP + 128
CH = 96
ZPT = ACC_ROWS // 16
WPT = HALF // 16

_mesh = plsc.VectorSubcoreMesh(core_axis_name="c", subcore_axis_name="s")
_sc_params = pltpu.CompilerParams(
    needs_layout_passes=False, use_tc_tiling_on_sc=False)


@functools.partial(
    pl.kernel,
    out_type=jax.ShapeDtypeStruct((NP, D), jnp.float32),
    mesh=_mesh,
    scratch_types=[
        pltpu.VMEM((SUP,), jnp.int32),
        pltpu.VMEM((SUP,), jnp.int32),
        pltpu.VMEM((SUP,), jnp.float32),
        pltpu.VMEM((CAP,), jnp.int32),
        pltpu.VMEM((CAP,), jnp.int32),
        pltpu.VMEM((CAP,), jnp.float32),
        pltpu.VMEM((CH, D), jnp.float32),
        pltpu.VMEM((CH, D), jnp.float32),
        pltpu.VMEM((CH,), jnp.int32),
        pltpu.VMEM((CH,), jnp.int32),
        pltpu.VMEM_SHARED((ACC_ROWS, D), jnp.float32),
        pltpu.SemaphoreType.DMA,
        pltpu.SemaphoreType.DMA,
        pltpu.SemaphoreType.DMA,
        pltpu.SemaphoreType.DMA,
        pltpu.SemaphoreType.DMA,
    ],
    compiler_params=_sc_params,
)
def _layer(x_hbm, src_hbm, dst_hbm, w_hbm, y_hbm,
           srcb, dstb, wb, srcc, dlocc, wc, rowsa, rowsb, ixa, ixb,
           acc, seml, semga, semgb, semsa, semsb):
    sc = lax.axis_index("c")
    sub = lax.axis_index("s")
    half_base = sc * HALF
    zero16 = jnp.zeros((16,), jnp.float32)

    def zrow(i, _):
        for q in range(4):
            rowsa[i, pl.ds(q * 16, 16)] = zero16
        return 0
    lax.fori_loop(0, CH, zrow, 0)
    zbase = sub * ZPT
    def zcopy(c, _):
        pltpu.sync_copy(rowsa.at[pl.ds(0, CH)],
                        acc.at[pl.ds(zbase + c * CH, CH)])
        return 0
    lax.fori_loop(0, ZPT // CH, zcopy, 0)
    pltpu.sync_copy(rowsa.at[pl.ds(0, ZPT % CH)],
                    acc.at[pl.ds(zbase + (ZPT // CH) * CH, ZPT % CH)])
    plsc.subcore_barrier()

    def sup_body(s, _):
        sbase = sub * EPT + s * SUP
        l1 = pltpu.async_copy(src_hbm.at[pl.ds(sbase, SUP)], srcb, seml)
        l2 = pltpu.async_copy(dst_hbm.at[pl.ds(sbase, SUP)], dstb, seml)
        l3 = pltpu.async_copy(w_hbm.at[pl.ds(sbase, SUP)], wb, seml)
        l1.wait()
        l2.wait()
        l3.wait()

        def grp(i, cnt):
            sl = pl.ds(i * 16, 16)
            t = dstb[sl] - half_base
            ok = (t >= 0) & (t < HALF)
            plsc.store_compressed(srcc.at[pl.ds(cnt, 16)], srcb[sl], mask=ok)
            plsc.store_compressed(dlocc.at[pl.ds(cnt, 16)], t, mask=ok)
            plsc.store_compressed(wc.at[pl.ds(cnt, 16)], wb[sl], mask=ok)
            return cnt + jnp.sum(ok.astype(jnp.int32))
        cnt = lax.fori_loop(0, NGRP, grp, jnp.int32(0))

        dummy16 = jnp.full((16,), HALF, jnp.int32)
        zrow16 = jnp.zeros((16,), jnp.int32)
        for k in range(CH // 16):
            srcc[pl.ds(cnt + k * 16, 16)] = zrow16
            dlocc[pl.ds(cnt + k * 16, 16)] = dummy16
            wc[pl.ds(cnt + k * 16, 16)] = zero16

        trip = lax.div(cnt + (CH - 1), jnp.int32(CH))
        pairs = lax.div(trip + 1, jnp.int32(2))

        def pair(p, _):
            off0 = p * (2 * CH)
            off1 = off0 + CH
            has1 = off1 < cnt
            da = pltpu.async_copy(
                x_hbm.at[srcc.at[pl.ds(off0, CH)]], rowsa, semga)
            @pl.when(has1)
            def _():
                pltpu.async_copy(
                    x_hbm.at[srcc.at[pl.ds(off1, CH)]], rowsb, semgb)
            da.wait()

            def scale(rows, off):
                @plsc.parallel_loop(0, CH, unroll=2)
                def _(e):
                    w16 = plsc.load_gather(
                        wc, [jnp.full((16,), off + e, jnp.int32)])
                    for q in range(4):
                        sl = pl.ds(q * 16, 16)
                        rows[e, sl] = rows[e, sl] * w16

            def stage_idx(ix, off):
                for k in range(CH // 16):
                    ix[pl.ds(k * 16, 16)] = dlocc[pl.ds(off + k * 16, 16)]

            scale(rowsa, off0)
            stage_idx(ixa, off0)
            sa = pltpu.async_copy(rowsa, acc.at[ixa], semsa, add=True)

            @pl.when(has1)
            def _():
                pltpu.make_async_copy(
                    x_hbm.at[srcc.at[pl.ds(off1, CH)]], rowsb, semgb).wait()
                scale(rowsb, off1)
                stage_idx(ixb, off1)
                pltpu.async_copy(rowsb, acc.at[ixb], semsb, add=True)

            sa.wait()
            @pl.when(has1)
            def _():
                pltpu.make_async_copy(rowsb, acc.at[ixb], semsb).wait()
            return 0
        lax.fori_loop(0, pairs, pair, 0)
        return 0
    lax.fori_loop(0, NSUP, sup_body, 0)
    plsc.subcore_barrier()

    wbase_l = sub * WPT
    wbase_g = half_base + wbase_l
    def wchunk(c, _):
        pltpu.sync_copy(acc.at[pl.ds(wbase_l + c * CH, CH)],
                        rowsa.at[pl.ds(0, CH)])
        pltpu.sync_copy(rowsa.at[pl.ds(0, CH)],
                        y_hbm.at[pl.ds(wbase_g + c * CH, CH)])
        return 0
    lax.fori_loop(0, WPT // CH, wchunk, 0)
    pltpu.sync_copy(acc.at[pl.ds(wbase_l + (WPT // CH) * CH, WPT % CH)],
                    rowsa.at[pl.ds(0, WPT % CH)])
    pltpu.sync_copy(rowsa.at[pl.ds(0, WPT % CH)],
                    y_hbm.at[pl.ds(wbase_g + (WPT // CH) * CH, WPT % CH)])


@functools.partial(
    pl.kernel,
    out_type=jax.ShapeDtypeStruct((B, D), jnp.float32),
    mesh=_mesh,
    scratch_types=[
        pltpu.VMEM((B // 32,), jnp.int32),
        pltpu.VMEM((B // 32, D), jnp.float32),
        pltpu.SemaphoreType.DMA,
    ],
    compiler_params=_sc_params,
)
def _gather_users(s_hbm, users_hbm, out_hbm, idxv, rowsv, sem):
    wid = lax.axis_index("s") * 2 + lax.axis_index("c")
    base = wid * (B // 32)
    pltpu.sync_copy(users_hbm.at[pl.ds(base, B // 32)], idxv)
    pltpu.async_copy(s_hbm.at[idxv], rowsv, sem).wait()
    pltpu.sync_copy(rowsv, out_hbm.at[pl.ds(base, B // 32)])


def _sum4_body(a, b, c, d, o):
    o[...] = (a[...] + b[...]) + (c[...] + d[...])


_sum4 = pl.pallas_call(
    _sum4_body,
    out_shape=jax.ShapeDtypeStruct((NP, D), jnp.float32),
    grid=(8,),
    in_specs=[pl.BlockSpec((NP // 8, D), lambda j: (j, 0))] * 4,
    out_specs=pl.BlockSpec((NP // 8, D), lambda j: (j, 0)),
)


UROW = 128


def _rating_body(u_ref, t_ref, o_ref):
    acc = lax.dot_general(u_ref[...], t_ref[...], (((1,), (1,)), ((), ())),
                          preferred_element_type=jnp.float32)
    o_ref[...] = jax.nn.sigmoid(acc * (1.0 / (N_LAYERS + 1) ** 2))


_rating = pl.pallas_call(
    _rating_body,
    out_shape=jax.ShapeDtypeStruct((B, NUM_ITEMS), jnp.float32),
    grid=(B // UROW,),
    in_specs=[
        pl.BlockSpec((UROW, D), lambda j: (j, 0)),
        pl.BlockSpec((NUM_ITEMS, D), lambda j: (0, 0)),
    ],
    out_specs=pl.BlockSpec((UROW, NUM_ITEMS), lambda j: (j, 0)),
)


def kernel(user_emb, item_emb, edge_index, edge_weight, users):
    x0 = jnp.concatenate(
        [user_emb, item_emb, jnp.zeros((NP - N, D), jnp.float32)], axis=0)
    src = edge_index[0]
    dst = edge_index[1]
    xs = [x0]
    for _ in range(N_LAYERS):
        xs.append(_layer(xs[-1], src, dst, edge_weight))
    s = _sum4(xs[0], xs[1], xs[2], xs[3])
    u = _gather_users(s, users)
    return _rating(u, s[NUM_USERS:N])
